# baseline (speedup 1.0000x reference)
import functools

import jax
import jax.numpy as jnp
from jax import lax
from jax.experimental import pallas as pl
from jax.experimental.pallas import tpu as pltpu
from jax.experimental.pallas import tpu_sc as plsc

B, L, D = 4096, 200, 128
N = B * L

NW = 32
ROWS_PW = N // NW
C = 64
NSTEPS = ROWS_PW // C


@functools.partial(
    pl.kernel,
    mesh=plsc.VectorSubcoreMesh(core_axis_name="c", subcore_axis_name="s"),
    out_type=jax.ShapeDtypeStruct((N, D), jnp.float32),
    scratch_types=[
        pltpu.VMEM((ROWS_PW,), jnp.int32),
        pltpu.VMEM((C, D), jnp.float32),
        pltpu.VMEM((C, D), jnp.float32),
        pltpu.VMEM_SHARED((2, D), jnp.float32),
        pltpu.SemaphoreType.DMA,
        pltpu.SemaphoreType.DMA,
        pltpu.SemaphoreType.DMA,
    ],
)
def _sc_lookup(table_hbm, idx_hbm, out_hbm, idx_all, r0, r1, tab_v, sem_g, so0, so1):
    wid = lax.axis_index("s") * 2 + lax.axis_index("c")
    base = wid * ROWS_PW
    row_bufs = (r0, r1)
    sems_out = (so0, so1)
    pltpu.sync_copy(table_hbm, tab_v)
    pltpu.sync_copy(idx_hbm.at[pl.ds(base, ROWS_PW)], idx_all)

    def fill(b, off):
        pltpu.async_copy(
            tab_v.at[idx_all.at[pl.ds(off - base, C)]], row_bufs[b], sem_g
        ).wait()

    def start_store(b, off):
        pltpu.async_copy(row_bufs[b], out_hbm.at[pl.ds(off, C)], sems_out[b])

    def wait_store(b, off):
        pltpu.make_async_copy(
            row_bufs[b], out_hbm.at[pl.ds(off, C)], sems_out[b]
        ).wait()

    for b in (0, 1):
        fill(b, base + b * C)
        start_store(b, base + b * C)

    def step(jj, carry):
        off2 = base + jj * 2 * C
        for b in (0, 1):
            off = off2 + b * C
            wait_store(b, off - 2 * C)
            fill(b, off)
            start_store(b, off)
        return carry

    lax.fori_loop(1, NSTEPS // 2, step, 0)
    for b in (0, 1):
        wait_store(b, base + (NSTEPS - 2 + b) * C)


def kernel(is_target_mask, embedding_weight):
    idx = is_target_mask.astype(jnp.int32).reshape(N)
    out = _sc_lookup(embedding_weight, idx)
    return out.reshape(B, L, D)


# speedup vs baseline: 1.0010x; 1.0010x over previous
import functools

import jax
import jax.numpy as jnp
from jax import lax
from jax.experimental import pallas as pl
from jax.experimental.pallas import tpu as pltpu
from jax.experimental.pallas import tpu_sc as plsc

B, L, D = 4096, 200, 128
N = B * L

NW = 32
ROWS_PW = N // NW
C = 128
NBUF = 4
NSTEPS = ROWS_PW // C


@functools.partial(
    pl.kernel,
    mesh=plsc.VectorSubcoreMesh(core_axis_name="c", subcore_axis_name="s"),
    out_type=jax.ShapeDtypeStruct((N, D), jnp.float32),
    scratch_types=[
        pltpu.VMEM((ROWS_PW,), jnp.int32),
        pltpu.VMEM((C, D), jnp.float32),
        pltpu.VMEM((C, D), jnp.float32),
        pltpu.VMEM((C, D), jnp.float32),
        pltpu.VMEM((C, D), jnp.float32),
        pltpu.VMEM_SHARED((2, D), jnp.float32),
        pltpu.SemaphoreType.DMA,
        pltpu.SemaphoreType.DMA,
        pltpu.SemaphoreType.DMA,
        pltpu.SemaphoreType.DMA,
        pltpu.SemaphoreType.DMA,
    ],
)
def _sc_lookup(
    table_hbm, idx_hbm, out_hbm, idx_all, r0, r1, r2, r3, tab_v, sem_g,
    so0, so1, so2, so3,
):
    wid = lax.axis_index("s") * 2 + lax.axis_index("c")
    base = wid * ROWS_PW
    row_bufs = (r0, r1, r2, r3)
    sems_out = (so0, so1, so2, so3)
    pltpu.sync_copy(table_hbm, tab_v)
    pltpu.sync_copy(idx_hbm.at[pl.ds(base, ROWS_PW)], idx_all)

    def fill(b, off):
        pltpu.async_copy(
            tab_v.at[idx_all.at[pl.ds(off - base, C)]], row_bufs[b], sem_g
        ).wait()

    def start_store(b, off):
        pltpu.async_copy(row_bufs[b], out_hbm.at[pl.ds(off, C)], sems_out[b])

    def wait_store(b, off):
        pltpu.make_async_copy(
            row_bufs[b], out_hbm.at[pl.ds(off, C)], sems_out[b]
        ).wait()

    for b in range(NBUF):
        fill(b, base + b * C)
        start_store(b, base + b * C)

    def step(jj, carry):
        offg = base + jj * NBUF * C
        for b in range(NBUF):
            off = offg + b * C
            wait_store(b, off - NBUF * C)
            fill(b, off)
            start_store(b, off)
        return carry

    lax.fori_loop(1, NSTEPS // NBUF, step, 0)
    for b in range(NBUF):
        wait_store(b, base + (NSTEPS - NBUF + b) * C)


def kernel(is_target_mask, embedding_weight):
    idx = is_target_mask.astype(jnp.int32).reshape(N)
    out = _sc_lookup(embedding_weight, idx)
    return out.reshape(B, L, D)


# speedup vs baseline: 1.0359x; 1.0349x over previous
import functools

import jax
import jax.numpy as jnp
from jax import lax
from jax.experimental import pallas as pl
from jax.experimental.pallas import tpu as pltpu
from jax.experimental.pallas import tpu_sc as plsc

B, L, D = 4096, 200, 128
N = B * L

NW = 32
ROWS_PW = N // NW
C = 128
NBUF = 4
NSTEPS = ROWS_PW // C


@functools.partial(
    pl.kernel,
    mesh=plsc.VectorSubcoreMesh(core_axis_name="c", subcore_axis_name="s"),
    out_type=jax.ShapeDtypeStruct((N, D), jnp.float32),
    scratch_types=[
        pltpu.VMEM((ROWS_PW,), jnp.int32),
        pltpu.VMEM((C, D), jnp.float32),
        pltpu.VMEM((C, D), jnp.float32),
        pltpu.VMEM((C, D), jnp.float32),
        pltpu.VMEM((C, D), jnp.float32),
        pltpu.VMEM_SHARED((2, D), jnp.float32),
        pltpu.SemaphoreType.DMA,
        pltpu.SemaphoreType.DMA,
        pltpu.SemaphoreType.DMA,
        pltpu.SemaphoreType.DMA,
        pltpu.SemaphoreType.DMA,
    ],
)
def _sc_lookup(
    table_hbm, idx_hbm, out_hbm, idx_all, r0, r1, r2, r3, tab_v, sem_g,
    so0, so1, so2, so3,
):
    wid = lax.axis_index("s") * 2 + lax.axis_index("c")
    base = wid * ROWS_PW
    row_bufs = (r0, r1, r2, r3)
    sems_out = (so0, so1, so2, so3)
    pltpu.sync_copy(table_hbm, tab_v)
    pltpu.sync_copy(idx_hbm.at[pl.ds(base, ROWS_PW)], idx_all)

    def start_fill(b, off):
        pltpu.async_copy(
            tab_v.at[idx_all.at[pl.ds(off - base, C)]], row_bufs[b], sem_g
        )

    def wait_fill(b, off):
        pltpu.make_async_copy(
            tab_v.at[idx_all.at[pl.ds(off - base, C)]], row_bufs[b], sem_g
        ).wait()

    def start_store(b, off):
        pltpu.async_copy(row_bufs[b], out_hbm.at[pl.ds(off, C)], sems_out[b])

    def wait_store(b, off):
        pltpu.make_async_copy(
            row_bufs[b], out_hbm.at[pl.ds(off, C)], sems_out[b]
        ).wait()

    start_fill(0, base)
    start_fill(1, base + C)
    for j in (0, 1):
        b = j % NBUF
        wait_fill(b, base + j * C)
        start_store(b, base + j * C)
        start_fill((j + 2) % NBUF, base + (j + 2) * C)

    def step(jj, carry):
        offg = base + jj * NBUF * C
        for k in range(NBUF):
            j_off = offg + (2 + k) * C
            b = (2 + k) % NBUF
            wait_fill(b, j_off)
            start_store(b, j_off)
            b2 = (2 + k + 2) % NBUF
            wait_store(b2, j_off - 2 * C)
            start_fill(b2, j_off + 2 * C)
        return carry

    lax.fori_loop(0, (NSTEPS - 4) // NBUF, step, 0)
    for j in (NSTEPS - 2, NSTEPS - 1):
        b = j % NBUF
        wait_fill(b, base + j * C)
        start_store(b, base + j * C)
    for b in range(NBUF):
        wait_store(b, base + (NSTEPS - NBUF + b) * C)


def kernel(is_target_mask, embedding_weight):
    idx = is_target_mask.astype(jnp.int32).reshape(N)
    out = _sc_lookup(embedding_weight, idx)
    return out.reshape(B, L, D)


# speedup vs baseline: 1.0952x; 1.0572x over previous
import functools

import jax
import jax.numpy as jnp
from jax import lax
from jax.experimental import pallas as pl
from jax.experimental.pallas import tpu as pltpu
from jax.experimental.pallas import tpu_sc as plsc

B, L, D = 4096, 200, 128
N = B * L

NW = 32
ROWS_PW = N // NW
C = 64
NBUF = 4
NSTEPS = ROWS_PW // C


@functools.partial(
    pl.kernel,
    mesh=plsc.VectorSubcoreMesh(core_axis_name="c", subcore_axis_name="s"),
    out_type=jax.ShapeDtypeStruct((N, D), jnp.float32),
    scratch_types=[
        pltpu.VMEM((ROWS_PW,), jnp.int32),
        pltpu.VMEM((C, D), jnp.float32),
        pltpu.VMEM((C, D), jnp.float32),
        pltpu.VMEM((C, D), jnp.float32),
        pltpu.VMEM((C, D), jnp.float32),
        pltpu.VMEM_SHARED((2, D), jnp.float32),
        pltpu.SemaphoreType.DMA,
        pltpu.SemaphoreType.DMA,
        pltpu.SemaphoreType.DMA,
        pltpu.SemaphoreType.DMA,
        pltpu.SemaphoreType.DMA,
    ],
)
def _sc_lookup(
    table_hbm, idx_hbm, out_hbm, idx_all, r0, r1, r2, r3, tab_v, sem_g,
    so0, so1, so2, so3,
):
    wid = lax.axis_index("s") * 2 + lax.axis_index("c")
    base = wid * ROWS_PW
    row_bufs = (r0, r1, r2, r3)
    sems_out = (so0, so1, so2, so3)
    pltpu.sync_copy(table_hbm, tab_v)
    pltpu.sync_copy(idx_hbm.at[pl.ds(base, ROWS_PW)], idx_all)

    def start_fill(b, off):
        pltpu.async_copy(
            tab_v.at[idx_all.at[pl.ds(off - base, C)]], row_bufs[b], sem_g
        )

    def wait_fill(b, off):
        pltpu.make_async_copy(
            tab_v.at[idx_all.at[pl.ds(off - base, C)]], row_bufs[b], sem_g
        ).wait()

    def start_store(b, off):
        pltpu.async_copy(row_bufs[b], out_hbm.at[pl.ds(off, C)], sems_out[b])

    def wait_store(b, off):
        pltpu.make_async_copy(
            row_bufs[b], out_hbm.at[pl.ds(off, C)], sems_out[b]
        ).wait()

    start_fill(0, base)
    start_fill(1, base + C)
    for j in (0, 1):
        b = j % NBUF
        wait_fill(b, base + j * C)
        start_store(b, base + j * C)
        start_fill((j + 2) % NBUF, base + (j + 2) * C)

    def step(jj, carry):
        offg = base + jj * NBUF * C
        for k in range(NBUF):
            j_off = offg + (2 + k) * C
            b = (2 + k) % NBUF
            wait_fill(b, j_off)
            start_store(b, j_off)
            b2 = (2 + k + 2) % NBUF
            wait_store(b2, j_off - 2 * C)
            start_fill(b2, j_off + 2 * C)
        return carry

    lax.fori_loop(0, (NSTEPS - 4) // NBUF, step, 0)
    for j in (NSTEPS - 2, NSTEPS - 1):
        b = j % NBUF
        wait_fill(b, base + j * C)
        start_store(b, base + j * C)
    for b in range(NBUF):
        wait_store(b, base + (NSTEPS - NBUF + b) * C)


def kernel(is_target_mask, embedding_weight):
    idx = is_target_mask.astype(jnp.int32).reshape(N)
    out = _sc_lookup(embedding_weight, idx)
    return out.reshape(B, L, D)


# speedup vs baseline: 1.1068x; 1.0106x over previous
import functools

import jax
import jax.numpy as jnp
from jax import lax
from jax.experimental import pallas as pl
from jax.experimental.pallas import tpu as pltpu
from jax.experimental.pallas import tpu_sc as plsc

B, L, D = 4096, 200, 128
N = B * L

NW = 32
ROWS_PW = N // NW
C = 32
NBUF = 4
NSTEPS = ROWS_PW // C


@functools.partial(
    pl.kernel,
    mesh=plsc.VectorSubcoreMesh(core_axis_name="c", subcore_axis_name="s"),
    out_type=jax.ShapeDtypeStruct((N, D), jnp.float32),
    scratch_types=[
        pltpu.VMEM((ROWS_PW,), jnp.int32),
        pltpu.VMEM((C, D), jnp.float32),
        pltpu.VMEM((C, D), jnp.float32),
        pltpu.VMEM((C, D), jnp.float32),
        pltpu.VMEM((C, D), jnp.float32),
        pltpu.VMEM_SHARED((2, D), jnp.float32),
        pltpu.SemaphoreType.DMA,
        pltpu.SemaphoreType.DMA,
        pltpu.SemaphoreType.DMA,
        pltpu.SemaphoreType.DMA,
        pltpu.SemaphoreType.DMA,
    ],
)
def _sc_lookup(
    table_hbm, idx_hbm, out_hbm, idx_all, r0, r1, r2, r3, tab_v, sem_g,
    so0, so1, so2, so3,
):
    wid = lax.axis_index("s") * 2 + lax.axis_index("c")
    base = wid * ROWS_PW
    row_bufs = (r0, r1, r2, r3)
    sems_out = (so0, so1, so2, so3)
    pltpu.sync_copy(table_hbm, tab_v)
    pltpu.sync_copy(idx_hbm.at[pl.ds(base, ROWS_PW)], idx_all)

    def start_fill(b, off):
        pltpu.async_copy(
            tab_v.at[idx_all.at[pl.ds(off - base, C)]], row_bufs[b], sem_g
        )

    def wait_fill(b, off):
        pltpu.make_async_copy(
            tab_v.at[idx_all.at[pl.ds(off - base, C)]], row_bufs[b], sem_g
        ).wait()

    def start_store(b, off):
        pltpu.async_copy(row_bufs[b], out_hbm.at[pl.ds(off, C)], sems_out[b])

    def wait_store(b, off):
        pltpu.make_async_copy(
            row_bufs[b], out_hbm.at[pl.ds(off, C)], sems_out[b]
        ).wait()

    start_fill(0, base)
    start_fill(1, base + C)
    for j in (0, 1):
        b = j % NBUF
        wait_fill(b, base + j * C)
        start_store(b, base + j * C)
        start_fill((j + 2) % NBUF, base + (j + 2) * C)

    def step(jj, carry):
        offg = base + jj * NBUF * C
        for k in range(NBUF):
            j_off = offg + (2 + k) * C
            b = (2 + k) % NBUF
            wait_fill(b, j_off)
            start_store(b, j_off)
            b2 = (2 + k + 2) % NBUF
            wait_store(b2, j_off - 2 * C)
            start_fill(b2, j_off + 2 * C)
        return carry

    lax.fori_loop(0, (NSTEPS - 4) // NBUF, step, 0)
    for j in (NSTEPS - 2, NSTEPS - 1):
        b = j % NBUF
        wait_fill(b, base + j * C)
        start_store(b, base + j * C)
    for b in range(NBUF):
        wait_store(b, base + (NSTEPS - NBUF + b) * C)


def kernel(is_target_mask, embedding_weight):
    idx = is_target_mask.astype(jnp.int32).reshape(N)
    out = _sc_lookup(embedding_weight, idx)
    return out.reshape(B, L, D)
